# baseline (speedup 1.0000x reference)
import functools

import jax
import jax.numpy as jnp
from jax import lax
from jax.experimental import pallas as pl
from jax.experimental.pallas import tpu as pltpu
from jax.experimental.pallas import tpu_sc as plsc

NUM_CORES = 2
NUM_SUBCORES = 16
NUM_WORKERS = NUM_CORES * NUM_SUBCORES
CHUNK = 64


@jax.jit
def _embed_lookup(ids_flat, token_embedding, pos_flat):
    n_rows = ids_flat.shape[0]
    seq_len, embed = pos_flat.shape
    n_batch = n_rows // seq_len
    groups_per_row = embed // 16

    mesh = plsc.VectorSubcoreMesh(
        core_axis_name="c", subcore_axis_name="s",
        num_cores=NUM_CORES, num_subcores=NUM_SUBCORES,
    )

    @functools.partial(
        pl.kernel,
        out_type=jax.ShapeDtypeStruct((n_rows, embed), jnp.float32),
        mesh=mesh,
        scratch_types=[
            pltpu.VMEM((CHUNK,), jnp.int32),
            pltpu.VMEM((CHUNK, embed), jnp.float32),
            pltpu.VMEM((CHUNK // 2, embed), jnp.float32),
            pltpu.SemaphoreType.DMA,
        ],
    )
    def emb_kernel(ids_hbm, tab_hbm, pos_hbm, out_hbm, idx_v, pos_v, buf, sem):
        half = CHUNK // 2
        wid = lax.axis_index("s") * NUM_CORES + lax.axis_index("c")
        s0 = wid * CHUNK
        pltpu.sync_copy(pos_hbm.at[pl.ds(s0, CHUNK)], pos_v)
        for b in range(n_batch):
            pltpu.sync_copy(ids_hbm.at[pl.ds(b * seq_len + s0, CHUNK)], idx_v)
            for h in range(2):
                r0 = b * seq_len + s0 + h * half
                pltpu.async_copy(
                    tab_hbm.at[idx_v.at[pl.ds(h * half, half)]], buf, sem
                ).wait()

                @plsc.parallel_loop(0, half * groups_per_row, unroll=8)
                def add_body(i):
                    r = i // groups_per_row
                    g = lax.rem(i, groups_per_row) * 16
                    plsc.addupdate(
                        buf.at[r, pl.ds(g, 16)],
                        pos_v[h * half + r, pl.ds(g, 16)],
                    )

                pltpu.sync_copy(buf, out_hbm.at[pl.ds(r0, half)])

    return emb_kernel(ids_flat, token_embedding, pos_flat)


def kernel(input_ids, token_embedding, position_embeds):
    b, s = input_ids.shape
    embed = token_embedding.shape[1]
    ids_flat = input_ids.astype(jnp.int32).reshape(b * s)
    pos_flat = position_embeds[0, :s, :]
    out = _embed_lookup(ids_flat, token_embedding, pos_flat)
    return out.reshape(b, s, embed)
